# baseline (speedup 1.0000x reference)
import functools

import jax
import jax.numpy as jnp
from jax import lax
from jax.experimental import pallas as pl
from jax.experimental.pallas import tpu as pltpu
from jax.experimental.pallas import tpu_sc as plsc

NC = 2
NS = 16
SUB = 128
KSUB = 4
MACRO = SUB * KSUB

N_DST = 50000
ROWS_PER_TILE = 3128
TRASH = NS * ROWS_PER_TILE
N_ACC = TRASH + 16


def _pad_edges(ei):
    E = ei.shape[1]
    ept = ((E // NS + MACRO - 1) // MACRO) * MACRO
    Ep = ept * NS
    pad = Ep - E
    idx = jnp.arange(pad, dtype=jnp.int32) % 16
    src = jnp.concatenate([ei[0].astype(jnp.int32), idx])
    dst = jnp.concatenate([ei[1].astype(jnp.int32), TRASH + idx])
    return (src.reshape(Ep // SUB, SUB), dst.reshape(Ep // SUB, SUB),
            ept // MACRO)


def _make_sc_round(n_macros_list, with_counts):
    ntypes = len(n_macros_list)

    def body(*refs):
        k = 0
        tables = []
        srcs = []
        dsts = []
        for _ in range(ntypes):
            tables.append(refs[k]); srcs.append(refs[k + 1])
            dsts.append(refs[k + 2]); k += 3
        outs = list(refs[k:k + ntypes]); k += ntypes
        cnts = []
        if with_counts:
            cnts = list(refs[k:k + ntypes]); k += ntypes
        (acc, cacc, sbuf, dbuf, rows, zbuf, z1d, ones, gsem) = refs[k:]

        core = lax.axis_index("c")
        sid = lax.axis_index("s")
        tile_base = sid * ROWS_PER_TILE

        zv = jnp.zeros((16,), jnp.float32)

        def fill_z(i, _):
            zbuf[i, pl.ds(0, 16)] = zv
            zbuf[i, pl.ds(16, 16)] = zv
            return 0
        lax.fori_loop(0, 512, fill_z, 0)

        def fill_z1(i, _):
            z1d[pl.ds(i * 16, 16)] = zv
            return 0
        lax.fori_loop(0, 196, fill_z1, 0)
        for i in range(8):
            ones[pl.ds(i * 16, 16)] = jnp.ones((16,), jnp.float32)

        for t in range(ntypes):
            n_macros = n_macros_list[t]
            tbl, src2d, dst2d, out = tables[t], srcs[t], dsts[t], outs[t]
            for cidx in range(2):
                col = core * 64 + cidx * 32
                do_cnt = with_counts and cidx == 0

                for r in range(6):
                    pltpu.sync_copy(
                        zbuf, acc.at[pl.ds(tile_base + 512 * r, 512), :])
                pltpu.sync_copy(zbuf.at[pl.ds(0, 56), :],
                                acc.at[pl.ds(tile_base + 3072, 56), :])

                @pl.when(sid == 0)
                def _():
                    pltpu.sync_copy(zbuf.at[pl.ds(0, 16), :],
                                    acc.at[pl.ds(TRASH, 16), :])
                if do_cnt:
                    @pl.when(core == 0)
                    def _():
                        pltpu.sync_copy(z1d.at[pl.ds(0, ROWS_PER_TILE)],
                                        cacc.at[pl.ds(tile_base,
                                                      ROWS_PER_TILE)])

                        @pl.when(sid == 0)
                        def _():
                            pltpu.sync_copy(z1d.at[pl.ds(0, 16)],
                                            cacc.at[pl.ds(TRASH, 16)])
                plsc.subcore_barrier()

                rows_per_tile_e = n_macros * KSUB
                base_row = sid * rows_per_tile_e

                def macro_body(m, _):
                    r0 = base_row + m * KSUB
                    pltpu.sync_copy(src2d.at[pl.ds(r0, KSUB), :], sbuf)
                    pltpu.sync_copy(dst2d.at[pl.ds(r0, KSUB), :], dbuf)
                    descs = [
                        pltpu.async_copy(
                            tbl.at[sbuf.at[j], pl.ds(col, 32)],
                            rows.at[j], gsem)
                        for j in range(KSUB)
                    ]
                    for d in descs:
                        d.wait()
                    for j in range(KSUB):
                        pltpu.sync_copy(rows.at[j], acc.at[dbuf.at[j]],
                                        add=True)
                    if do_cnt:
                        @pl.when(core == 0)
                        def _():
                            for j in range(KSUB):
                                pltpu.sync_copy(ones, cacc.at[dbuf.at[j]],
                                                add=True)
                    return 0
                lax.fori_loop(0, n_macros, macro_body, 0)
                plsc.subcore_barrier()

                @pl.when(sid < NS - 1)
                def _():
                    pltpu.sync_copy(
                        acc.at[pl.ds(tile_base, ROWS_PER_TILE), :],
                        out.at[pl.ds(tile_base, ROWS_PER_TILE),
                               pl.ds(col, 32)])

                @pl.when(sid == NS - 1)
                def _():
                    last = N_DST - (NS - 1) * ROWS_PER_TILE
                    pltpu.sync_copy(
                        acc.at[pl.ds(tile_base, last), :],
                        out.at[pl.ds(tile_base, last), pl.ds(col, 32)])

                if do_cnt:
                    @pl.when(core == 0)
                    def _():
                        @pl.when(sid < NS - 1)
                        def _():
                            pltpu.sync_copy(
                                cacc.at[pl.ds(tile_base, ROWS_PER_TILE)],
                                cnts[t].at[pl.ds(tile_base,
                                                 ROWS_PER_TILE)])

                        @pl.when(sid == NS - 1)
                        def _():
                            last = N_DST - (NS - 1) * ROWS_PER_TILE
                            pltpu.sync_copy(
                                cacc.at[pl.ds(tile_base, last)],
                                cnts[t].at[pl.ds(tile_base, last)])

    out_type = [jax.ShapeDtypeStruct((N_DST, 128), jnp.float32)
                for _ in range(ntypes)]
    if with_counts:
        out_type += [jax.ShapeDtypeStruct((N_DST,), jnp.float32)
                     for _ in range(ntypes)]

    mesh = plsc.VectorSubcoreMesh(core_axis_name="c", subcore_axis_name="s",
                                  num_cores=NC, num_subcores=NS)
    return pl.kernel(
        body,
        out_type=out_type,
        mesh=mesh,
        compiler_params=pltpu.CompilerParams(use_tc_tiling_on_sc=False),
        scratch_types=[
            pltpu.VMEM_SHARED((N_ACC, 32), jnp.float32),
            pltpu.VMEM_SHARED((N_ACC,), jnp.float32),
            pltpu.VMEM((KSUB, SUB), jnp.int32),
            pltpu.VMEM((KSUB, SUB), jnp.int32),
            pltpu.VMEM((KSUB, SUB, 32), jnp.float32),
            pltpu.VMEM((512, 32), jnp.float32),
            pltpu.VMEM((3136,), jnp.float32),
            pltpu.VMEM((SUB,), jnp.float32),
            pltpu.SemaphoreType.DMA,
        ],
    )


BLK = 1000
GRID = N_DST // BLK


def _tc1_body(Sc, Sw, Sb, cc, cw, cb, xp, xa,
              Wlc, Wlw, Wlb, Wrc, Wrw, Wrb, bc, bw, bb, p1, a1):
    f32 = jnp.float32
    ic = 1.0 / jnp.maximum(cc[...], 1.0)
    iw = 1.0 / jnp.maximum(cw[...], 1.0)
    ib = 1.0 / jnp.maximum(cb[...], 1.0)
    dot = functools.partial(lax.dot_general,
                            dimension_numbers=(((1,), (1,)), ((), ())),
                            preferred_element_type=f32)
    p = dot(Sc[...] * ic, Wlc[...]) + dot(Sw[...] * iw, Wlw[...])
    p = p + dot(xp[...], Wrc[...] + Wrw[...])
    p1[...] = p + (bc[...] + bw[...])[None, :]
    a = dot(Sb[...] * ib, Wlb[...]) + dot(xa[...], Wrb[...])
    a1[...] = a + bb[...][None, :]


def _tc2_body(Sc, Sw, cc, cw, p1, Wlc, Wlw, Wrc, Wrw, bc, bw, Wh, bh, out):
    f32 = jnp.float32
    ic = 1.0 / jnp.maximum(cc[...], 1.0)
    iw = 1.0 / jnp.maximum(cw[...], 1.0)
    dot = functools.partial(lax.dot_general,
                            dimension_numbers=(((1,), (1,)), ((), ())),
                            preferred_element_type=f32)
    p2 = dot(Sc[...] * ic, Wlc[...]) + dot(Sw[...] * iw, Wlw[...])
    p2 = p2 + dot(p1[...], Wrc[...] + Wrw[...]) + (bc[...] + bw[...])[None, :]
    out[...] = dot(p2, Wh[...]) + bh[...][None, :]


def _row_spec():
    return pl.BlockSpec((BLK, 128), lambda i: (i, 0))


def _cnt_spec():
    return pl.BlockSpec((BLK, 1), lambda i: (i, 0))


def _full2(shape):
    return pl.BlockSpec(shape, lambda i: tuple(0 for _ in shape))


def kernel(x_paper, x_author, ei_cites, ei_writes, ei_written_by,
           Wl_c1, Wr_c1, b_c1, Wl_w1, Wr_w1, b_w1, Wl_b1, Wr_b1, b_b1,
           Wl_c2, Wr_c2, b_c2, Wl_w2, Wr_w2, b_w2, Wh, bh):
    def _seg(x, ei, n):
        s = jax.ops.segment_sum(jnp.take(x, ei[0], axis=0), ei[1],
                                num_segments=n)
        c = jax.ops.segment_sum(jnp.ones((ei.shape[1],), jnp.float32),
                                ei[1], num_segments=n)
        return s, c
    S_c1, cnt_c = _seg(x_paper, ei_cites, N_DST)
    S_w1, cnt_w = _seg(x_author, ei_writes, N_DST)
    S_b1, cnt_b = _seg(x_paper, ei_written_by, N_DST)

    cc = cnt_c.reshape(N_DST, 1)
    cw = cnt_w.reshape(N_DST, 1)
    cb = cnt_b.reshape(N_DST, 1)

    w_spec = _full2((128, 128))
    b_spec = pl.BlockSpec((128,), lambda i: (0,))
    p1, a1 = pl.pallas_call(
        _tc1_body,
        grid=(GRID,),
        in_specs=[_row_spec(), _row_spec(), _row_spec(),
                  _cnt_spec(), _cnt_spec(), _cnt_spec(),
                  _row_spec(), _row_spec(),
                  w_spec, w_spec, w_spec, w_spec, w_spec, w_spec,
                  b_spec, b_spec, b_spec],
        out_specs=[_row_spec(), _row_spec()],
        out_shape=[jax.ShapeDtypeStruct((N_DST, 128), jnp.float32),
                   jax.ShapeDtypeStruct((N_DST, 128), jnp.float32)],
    )(S_c1, S_w1, S_b1, cc, cw, cb, x_paper, x_author,
      Wl_c1, Wl_w1, Wl_b1, Wr_c1, Wr_w1, Wr_b1, b_c1, b_w1, b_b1)

    S_c2, _ = _seg(p1, ei_cites, N_DST)
    S_w2, _ = _seg(a1, ei_writes, N_DST)

    out = pl.pallas_call(
        _tc2_body,
        grid=(GRID,),
        in_specs=[_row_spec(), _row_spec(), _cnt_spec(), _cnt_spec(),
                  _row_spec(),
                  w_spec, w_spec, w_spec, w_spec,
                  pl.BlockSpec((128,), lambda i: (0,)),
                  pl.BlockSpec((128,), lambda i: (0,)),
                  _full2((349, 128)),
                  pl.BlockSpec((349,), lambda i: (0,))],
        out_specs=pl.BlockSpec((BLK, 349), lambda i: (i, 0)),
        out_shape=jax.ShapeDtypeStruct((N_DST, 349), jnp.float32),
    )(S_c2, S_w2, cc, cw, p1,
      Wl_c2, Wl_w2, Wr_c2, Wr_w2, b_c2, b_w2, Wh, bh)
    return out


# speedup vs baseline: 1.5296x; 1.5296x over previous
import functools

import jax
import jax.numpy as jnp
from jax import lax
from jax.experimental import pallas as pl
from jax.experimental.pallas import tpu as pltpu
from jax.experimental.pallas import tpu_sc as plsc

NC = 2
NS = 16
SUB = 128
KSUB = 1
MACRO = SUB * KSUB

N_DST = 50000
RANGE = 12512
NRANGE = 4
WR = 784
WR15 = 752
WR15_LAST = 704
ZBASE = N_DST
ZPAD = 512


def _pad_edges(ei):
    E = ei.shape[1]
    ept = ((E // NS + MACRO - 1) // MACRO) * MACRO
    Ep = ept * NS
    pad = Ep - E
    idx = jnp.arange(pad, dtype=jnp.int32)
    src = jnp.concatenate([ei[0].astype(jnp.int32), ZBASE + idx % ZPAD])
    dst = jnp.concatenate([ei[1].astype(jnp.int32), ZBASE + idx % 16])
    return (src.reshape(Ep // SUB, SUB), dst.reshape(Ep // SUB, SUB),
            ept // MACRO)


def _make_sc_round(n_macros_list, with_counts):
    ntypes = len(n_macros_list)

    def body(*refs):
        k = 0
        tables = []
        srcs = []
        dsts = []
        for _ in range(ntypes):
            tables.append(refs[k]); srcs.append(refs[k + 1])
            dsts.append(refs[k + 2]); k += 3
        outs = list(refs[k:k + ntypes]); k += ntypes
        cnts = []
        if with_counts:
            cnts = list(refs[k:k + ntypes]); k += ntypes
        (acc, cacc, sbuf, dbuf, s2buf, d2buf, vbuf, rows,
         zbuf, z1d, cbuf, gsem) = refs[k:]

        core = lax.axis_index("c")
        sid = lax.axis_index("s")
        zv = jnp.zeros((16,), jnp.float32)

        def fill_z(i, _):
            for kk in range(8):
                zbuf[i, pl.ds(kk * 16, 16)] = zv
            return 0
        lax.fori_loop(0, 64, fill_z, 0)

        def fill_z1(i, _):
            z1d[pl.ds(i * 16, 16)] = zv
            return 0
        lax.fori_loop(0, 49, fill_z1, 0)

        for t in range(ntypes):
            n_macros = n_macros_list[t]
            tbl, src2d, dst2d, out = tables[t], srcs[t], dsts[t], outs[t]
            for rr in range(2):
                rid = core * 2 + rr
                base = rid * RANGE
                lo_off = sid * WR

                @pl.when(sid < NS - 1)
                def _():
                    for r in range(12):
                        pltpu.sync_copy(
                            zbuf, acc.at[pl.ds(lo_off + 64 * r, 64), :])
                    pltpu.sync_copy(zbuf.at[pl.ds(0, 16), :],
                                    acc.at[pl.ds(lo_off + 768, 16), :])
                    if with_counts:
                        pltpu.sync_copy(z1d, cacc.at[pl.ds(lo_off, WR)])

                @pl.when(sid == NS - 1)
                def _():
                    for r in range(11):
                        pltpu.sync_copy(
                            zbuf, acc.at[pl.ds(lo_off + 64 * r, 64), :])
                    pltpu.sync_copy(zbuf.at[pl.ds(0, 48), :],
                                    acc.at[pl.ds(lo_off + 704, 48), :])
                    if with_counts:
                        pltpu.sync_copy(z1d.at[pl.ds(0, WR15)],
                                        cacc.at[pl.ds(lo_off, WR15)])
                plsc.subcore_barrier()

                erows = n_macros * KSUB
                base_row = sid * erows

                def macro_body(m, _):
                    r0 = base_row + m * KSUB
                    pltpu.sync_copy(src2d.at[pl.ds(r0, KSUB), :], sbuf)
                    pltpu.sync_copy(dst2d.at[pl.ds(r0, KSUB), :], dbuf)
                    for j in range(KSUB):
                        for kk in range(8):
                            sl = pl.ds(kk * 16, 16)
                            s = sbuf[j, sl]
                            d = dbuf[j, sl]
                            lov = d - base
                            inr = (lov >= 0) & (lov < RANGE)
                            s2buf[j, sl] = jnp.where(
                                inr, s, ZBASE + (d & (ZPAD - 1)))
                            d2buf[j, sl] = jnp.where(inr, lov, d & 8191)
                            if with_counts:
                                real = inr & (s < ZBASE)
                                vbuf[j, sl] = jnp.where(real, 1.0, 0.0)
                    descs = [
                        pltpu.async_copy(tbl.at[s2buf.at[j]], rows.at[j],
                                         gsem)
                        for j in range(KSUB)
                    ]
                    for d in descs:
                        d.wait()
                    for j in range(KSUB):
                        pltpu.sync_copy(rows.at[j], acc.at[d2buf.at[j]],
                                        add=True)
                    if with_counts:
                        for j in range(KSUB):
                            pltpu.sync_copy(vbuf.at[j],
                                            cacc.at[d2buf.at[j]], add=True)
                    return 0
                lax.fori_loop(0, n_macros, macro_body, 0)
                plsc.subcore_barrier()

                gbase = base + lo_off

                @pl.when(sid < NS - 1)
                def _():
                    pltpu.sync_copy(acc.at[pl.ds(lo_off, WR), :],
                                    out.at[pl.ds(gbase, WR), :])
                    if with_counts:
                        pltpu.sync_copy(cacc.at[pl.ds(lo_off, WR)], cbuf)
                        pltpu.sync_copy(cbuf,
                                        cnts[t].at[pl.ds(gbase, WR)])

                @pl.when((sid == NS - 1) & (rid < NRANGE - 1))
                def _():
                    pltpu.sync_copy(acc.at[pl.ds(lo_off, WR15), :],
                                    out.at[pl.ds(gbase, WR15), :])
                    if with_counts:
                        pltpu.sync_copy(cacc.at[pl.ds(lo_off, WR15)],
                                        cbuf.at[pl.ds(0, WR15)])
                        pltpu.sync_copy(cbuf.at[pl.ds(0, WR15)],
                                        cnts[t].at[pl.ds(gbase, WR15)])

                @pl.when((sid == NS - 1) & (rid == NRANGE - 1))
                def _():
                    pltpu.sync_copy(acc.at[pl.ds(lo_off, WR15_LAST), :],
                                    out.at[pl.ds(gbase, WR15_LAST), :])
                    if with_counts:
                        pltpu.sync_copy(cacc.at[pl.ds(lo_off, WR15_LAST)],
                                        cbuf.at[pl.ds(0, WR15_LAST)])
                        pltpu.sync_copy(
                            cbuf.at[pl.ds(0, WR15_LAST)],
                            cnts[t].at[pl.ds(gbase, WR15_LAST)])

    out_type = [jax.ShapeDtypeStruct((N_DST, 128), jnp.float32)
                for _ in range(ntypes)]
    if with_counts:
        out_type += [jax.ShapeDtypeStruct((N_DST,), jnp.float32)
                     for _ in range(ntypes)]

    mesh = plsc.VectorSubcoreMesh(core_axis_name="c", subcore_axis_name="s",
                                  num_cores=NC, num_subcores=NS)
    return pl.kernel(
        body,
        out_type=out_type,
        mesh=mesh,
        scratch_types=[
            pltpu.VMEM_SHARED((RANGE, 128), jnp.float32),
            pltpu.VMEM_SHARED((RANGE,), jnp.float32),
            pltpu.VMEM((KSUB, SUB), jnp.int32),
            pltpu.VMEM((KSUB, SUB), jnp.int32),
            pltpu.VMEM((KSUB, SUB), jnp.int32),
            pltpu.VMEM((KSUB, SUB), jnp.int32),
            pltpu.VMEM((KSUB, SUB), jnp.float32),
            pltpu.VMEM((KSUB, SUB, 128), jnp.float32),
            pltpu.VMEM((64, 128), jnp.float32),
            pltpu.VMEM((784,), jnp.float32),
            pltpu.VMEM((784,), jnp.float32),
            pltpu.SemaphoreType.DMA,
        ],
    )


BLK = 1000
GRID = N_DST // BLK


def _tc1_body(Sc, Sw, Sb, cc, cw, cb, xp, xa,
              Wlc, Wlw, Wlb, Wrc, Wrw, Wrb, bc, bw, bb, p1, a1):
    f32 = jnp.float32
    ic = 1.0 / jnp.maximum(cc[...], 1.0)
    iw = 1.0 / jnp.maximum(cw[...], 1.0)
    ib = 1.0 / jnp.maximum(cb[...], 1.0)
    dot = functools.partial(lax.dot_general,
                            dimension_numbers=(((1,), (1,)), ((), ())),
                            preferred_element_type=f32)
    p = dot(Sc[...] * ic, Wlc[...]) + dot(Sw[...] * iw, Wlw[...])
    p = p + dot(xp[...], Wrc[...] + Wrw[...])
    p1[...] = p + (bc[...] + bw[...])[None, :]
    a = dot(Sb[...] * ib, Wlb[...]) + dot(xa[...], Wrb[...])
    a1[...] = a + bb[...][None, :]


def _tc2_body(Sc, Sw, cc, cw, p1, Wlc, Wlw, Wrc, Wrw, bc, bw, Wh, bh, out):
    f32 = jnp.float32
    ic = 1.0 / jnp.maximum(cc[...], 1.0)
    iw = 1.0 / jnp.maximum(cw[...], 1.0)
    dot = functools.partial(lax.dot_general,
                            dimension_numbers=(((1,), (1,)), ((), ())),
                            preferred_element_type=f32)
    p2 = dot(Sc[...] * ic, Wlc[...]) + dot(Sw[...] * iw, Wlw[...])
    p2 = p2 + dot(p1[...], Wrc[...] + Wrw[...]) + (bc[...] + bw[...])[None, :]
    out[...] = dot(p2, Wh[...]) + bh[...][None, :]


def _row_spec():
    return pl.BlockSpec((BLK, 128), lambda i: (i, 0))


def _cnt_spec():
    return pl.BlockSpec((BLK, 1), lambda i: (i, 0))


def _full2(shape):
    return pl.BlockSpec(shape, lambda i: tuple(0 for _ in shape))


def kernel(x_paper, x_author, ei_cites, ei_writes, ei_written_by,
           Wl_c1, Wr_c1, b_c1, Wl_w1, Wr_w1, b_w1, Wl_b1, Wr_b1, b_b1,
           Wl_c2, Wr_c2, b_c2, Wl_w2, Wr_w2, b_w2, Wh, bh):
    zrows = jnp.zeros((ZPAD, 128), jnp.float32)
    xp_t = jnp.concatenate([x_paper, zrows])
    xa_t = jnp.concatenate([x_author, zrows])
    srcC, dstC, nmC = _pad_edges(ei_cites)
    srcW, dstW, nmW = _pad_edges(ei_writes)
    srcB, dstB, nmB = _pad_edges(ei_written_by)

    round1 = _make_sc_round([nmC, nmW, nmB], with_counts=True)
    S_c1, S_w1, S_b1, cnt_c, cnt_w, cnt_b = round1(
        xp_t, srcC, dstC, xa_t, srcW, dstW, xp_t, srcB, dstB)

    cc = cnt_c.reshape(N_DST, 1)
    cw = cnt_w.reshape(N_DST, 1)
    cb = cnt_b.reshape(N_DST, 1)

    w_spec = _full2((128, 128))
    b_spec = pl.BlockSpec((128,), lambda i: (0,))
    p1, a1 = pl.pallas_call(
        _tc1_body,
        grid=(GRID,),
        in_specs=[_row_spec(), _row_spec(), _row_spec(),
                  _cnt_spec(), _cnt_spec(), _cnt_spec(),
                  _row_spec(), _row_spec(),
                  w_spec, w_spec, w_spec, w_spec, w_spec, w_spec,
                  b_spec, b_spec, b_spec],
        out_specs=[_row_spec(), _row_spec()],
        out_shape=[jax.ShapeDtypeStruct((N_DST, 128), jnp.float32),
                   jax.ShapeDtypeStruct((N_DST, 128), jnp.float32)],
    )(S_c1, S_w1, S_b1, cc, cw, cb, x_paper, x_author,
      Wl_c1, Wl_w1, Wl_b1, Wr_c1, Wr_w1, Wr_b1, b_c1, b_w1, b_b1)

    p1_t = jnp.concatenate([p1, zrows])
    a1_t = jnp.concatenate([a1, zrows])
    round2 = _make_sc_round([nmC, nmW], with_counts=False)
    S_c2, S_w2 = round2(p1_t, srcC, dstC, a1_t, srcW, dstW)

    out = pl.pallas_call(
        _tc2_body,
        grid=(GRID,),
        in_specs=[_row_spec(), _row_spec(), _cnt_spec(), _cnt_spec(),
                  _row_spec(),
                  w_spec, w_spec, w_spec, w_spec,
                  pl.BlockSpec((128,), lambda i: (0,)),
                  pl.BlockSpec((128,), lambda i: (0,)),
                  _full2((349, 128)),
                  pl.BlockSpec((349,), lambda i: (0,))],
        out_specs=pl.BlockSpec((BLK, 349), lambda i: (i, 0)),
        out_shape=jax.ShapeDtypeStruct((N_DST, 349), jnp.float32),
    )(S_c2, S_w2, cc, cw, p1,
      Wl_c2, Wl_w2, Wr_c2, Wr_w2, b_c2, b_w2, Wh, bh)
    return out


# speedup vs baseline: 4.0079x; 2.6201x over previous
import functools

import jax
import jax.numpy as jnp
from jax import lax
from jax.experimental import pallas as pl
from jax.experimental.pallas import tpu as pltpu
from jax.experimental.pallas import tpu_sc as plsc

NC = 2
NS = 16
SUB = 128
MACRO = 2048

N_DST = 50000
RANGE = 12512
NRANGE = 4
WR = 784
WR15 = 752
WR15_LAST = 704
ZBASE = N_DST
ZPAD = 512


def _pad_edges(ei):
    E = ei.shape[1]
    ept = ((E // NS + MACRO - 1) // MACRO) * MACRO
    Ep = ept * NS
    pad = Ep - E
    idx = jnp.arange(pad, dtype=jnp.int32)
    src = jnp.concatenate([ei[0].astype(jnp.int32), ZBASE + idx % ZPAD])
    dst = jnp.concatenate([ei[1].astype(jnp.int32),
                           NRANGE * RANGE + idx % 16])
    return src, dst, ept // MACRO


def _make_sc_round(n_macros_list, with_counts):
    ntypes = len(n_macros_list)

    def body(*refs):
        k = 0
        tables = []
        srcs = []
        dsts = []
        for _ in range(ntypes):
            tables.append(refs[k]); srcs.append(refs[k + 1])
            dsts.append(refs[k + 2]); k += 3
        outs = list(refs[k:k + ntypes]); k += ntypes
        cnts = []
        if with_counts:
            cnts = list(refs[k:k + ntypes]); k += ntypes
        (acc, cacc, sbuf, dbuf, ps, pd, pdrow, ones, vt, rows,
         zbuf, z1d, gsem) = refs[k:]

        core = lax.axis_index("c")
        sid = lax.axis_index("s")
        lane = lax.broadcasted_iota(jnp.int32, (16,), 0)
        zv = jnp.zeros((16,), jnp.float32)
        ov = jnp.ones((16,), jnp.float32)

        def fill_z(i, _):
            for kk in range(8):
                zbuf[i, pl.ds(kk * 16, 16)] = zv
            return 0
        lax.fori_loop(0, 16, fill_z, 0)

        def fill_z1(i, _):
            z1d[pl.ds(i * 16, 16)] = zv
            return 0
        lax.fori_loop(0, 49, fill_z1, 0)
        for i in range(8):
            ones[pl.ds(i * 16, 16)] = ov

        for t in range(ntypes):
            n_macros = n_macros_list[t]
            tbl, src1d, dst1d, out = tables[t], srcs[t], dsts[t], outs[t]
            for rr in range(2):
                rid = core * 2 + rr
                base = rid * RANGE
                lo_off = sid * WR

                @pl.when(sid < NS - 1)
                def _():
                    for r in range(49):
                        pltpu.sync_copy(
                            zbuf, acc.at[pl.ds(lo_off + 16 * r, 16), :])
                    if with_counts:
                        pltpu.sync_copy(z1d, cacc.at[pl.ds(lo_off, WR)])

                @pl.when(sid == NS - 1)
                def _():
                    for r in range(47):
                        pltpu.sync_copy(
                            zbuf, acc.at[pl.ds(lo_off + 16 * r, 16), :])
                    if with_counts:
                        pltpu.sync_copy(z1d.at[pl.ds(0, WR15)],
                                        cacc.at[pl.ds(lo_off, WR15)])
                plsc.subcore_barrier()

                ept = n_macros * MACRO
                ebase = sid * ept

                def macro_body(m, _):
                    e0 = ebase + m * MACRO
                    pltpu.sync_copy(src1d.at[pl.ds(e0, MACRO)], sbuf)
                    pltpu.sync_copy(dst1d.at[pl.ds(e0, MACRO)], dbuf)

                    def scan_body(i, wp):
                        s = sbuf[pl.ds(i * 16, 16)]
                        d = dbuf[pl.ds(i * 16, 16)]
                        lov = d - base
                        inr = (lov >= 0) & (lov < RANGE)
                        pc = plsc.all_reduce_population_count(inr)
                        im = inr.astype(jnp.int32)
                        pos = wp + plsc.cumsum(im) - 1
                        plsc.store_scatter(ps, [pos], s, mask=inr)
                        plsc.store_scatter(pd, [pos], lov, mask=inr)
                        return wp + jnp.squeeze(
                            lax.slice(pc, (0,), (1,)))
                    wp = lax.fori_loop(0, MACRO // 16, scan_body, 0)
                    nb = wp // 128
                    rem = wp - nb * 128

                    @pl.when(rem > 0)
                    def _():
                        for ff in range(8):
                            addr = nb * 128 + ff * 16
                            fmask = (ff * 16 + lane) >= rem
                            cs = ps[pl.ds(addr, 16)]
                            cd = pd[pl.ds(addr, 16)]
                            fs = ZBASE + ((addr + lane) & (ZPAD - 1))
                            fd = (lo_off + ff * 16 + lane) & 4095
                            ps[pl.ds(addr, 16)] = jnp.where(fmask, fs, cs)
                            pd[pl.ds(addr, 16)] = jnp.where(fmask, fd, cd)
                            if with_counts:
                                vt[pl.ds(ff * 16, 16)] = jnp.where(
                                    fmask, 0.0, 1.0)

                    def fire(b, cnt_src):
                        rp = b * 128
                        for kk in range(8):
                            pdrow[0, pl.ds(kk * 16, 16)] = (
                                pd[pl.ds(rp + kk * 16, 16)])
                        pltpu.async_copy(
                            tbl.at[ps.at[pl.ds(rp, 128)]], rows,
                            gsem).wait()
                        pltpu.sync_copy(rows, acc.at[pdrow.at[0]],
                                        add=True)
                        if with_counts:
                            pltpu.sync_copy(cnt_src,
                                            cacc.at[pdrow.at[0]],
                                            add=True)

                    def fire_body(b, _):
                        fire(b, ones)
                        return 0
                    lax.fori_loop(0, nb, fire_body, 0)

                    @pl.when(rem > 0)
                    def _():
                        fire(nb, vt)
                    return 0
                lax.fori_loop(0, n_macros, macro_body, 0)
                plsc.subcore_barrier()

                gbase = base + lo_off

                @pl.when(sid < NS - 1)
                def _():
                    pltpu.sync_copy(acc.at[pl.ds(lo_off, WR), :],
                                    out.at[pl.ds(gbase, WR), :])
                    if with_counts:
                        pltpu.sync_copy(cacc.at[pl.ds(lo_off, WR)], z1d)
                        pltpu.sync_copy(z1d,
                                        cnts[t].at[pl.ds(gbase, WR)])

                @pl.when((sid == NS - 1) & (rid < NRANGE - 1))
                def _():
                    pltpu.sync_copy(acc.at[pl.ds(lo_off, WR15), :],
                                    out.at[pl.ds(gbase, WR15), :])
                    if with_counts:
                        pltpu.sync_copy(cacc.at[pl.ds(lo_off, WR15)],
                                        z1d.at[pl.ds(0, WR15)])
                        pltpu.sync_copy(z1d.at[pl.ds(0, WR15)],
                                        cnts[t].at[pl.ds(gbase, WR15)])

                @pl.when((sid == NS - 1) & (rid == NRANGE - 1))
                def _():
                    pltpu.sync_copy(acc.at[pl.ds(lo_off, WR15_LAST), :],
                                    out.at[pl.ds(gbase, WR15_LAST), :])
                    if with_counts:
                        pltpu.sync_copy(
                            cacc.at[pl.ds(lo_off, WR15_LAST)],
                            z1d.at[pl.ds(0, WR15_LAST)])
                        pltpu.sync_copy(
                            z1d.at[pl.ds(0, WR15_LAST)],
                            cnts[t].at[pl.ds(gbase, WR15_LAST)])

                if with_counts:
                    lax.fori_loop(0, 49, fill_z1, 0)

    out_type = [jax.ShapeDtypeStruct((N_DST, 128), jnp.float32)
                for _ in range(ntypes)]
    if with_counts:
        out_type += [jax.ShapeDtypeStruct((N_DST,), jnp.float32)
                     for _ in range(ntypes)]

    mesh = plsc.VectorSubcoreMesh(core_axis_name="c", subcore_axis_name="s",
                                  num_cores=NC, num_subcores=NS)
    return pl.kernel(
        body,
        out_type=out_type,
        mesh=mesh,
        compiler_params=pltpu.CompilerParams(needs_layout_passes=False),
        scratch_types=[
            pltpu.VMEM_SHARED((RANGE, 128), jnp.float32),
            pltpu.VMEM_SHARED((RANGE,), jnp.float32),
            pltpu.VMEM((MACRO,), jnp.int32),
            pltpu.VMEM((MACRO,), jnp.int32),
            pltpu.VMEM((MACRO + 144,), jnp.int32),
            pltpu.VMEM((MACRO + 144,), jnp.int32),
            pltpu.VMEM((1, 128), jnp.int32),
            pltpu.VMEM((128,), jnp.float32),
            pltpu.VMEM((128,), jnp.float32),
            pltpu.VMEM((128, 128), jnp.float32),
            pltpu.VMEM((16, 128), jnp.float32),
            pltpu.VMEM((784,), jnp.float32),
            pltpu.SemaphoreType.DMA,
        ],
    )


BLK = 1000
GRID = N_DST // BLK


def _tc1_body(Sc, Sw, Sb, cc, cw, cb, xp, xa,
              Wlc, Wlw, Wlb, Wrc, Wrw, Wrb, bc, bw, bb, p1, a1):
    f32 = jnp.float32
    ic = 1.0 / jnp.maximum(cc[...], 1.0)
    iw = 1.0 / jnp.maximum(cw[...], 1.0)
    ib = 1.0 / jnp.maximum(cb[...], 1.0)
    dot = functools.partial(lax.dot_general,
                            dimension_numbers=(((1,), (1,)), ((), ())),
                            preferred_element_type=f32)
    p = dot(Sc[...] * ic, Wlc[...]) + dot(Sw[...] * iw, Wlw[...])
    p = p + dot(xp[...], Wrc[...] + Wrw[...])
    p1[...] = p + (bc[...] + bw[...])[None, :]
    a = dot(Sb[...] * ib, Wlb[...]) + dot(xa[...], Wrb[...])
    a1[...] = a + bb[...][None, :]


def _tc2_body(Sc, Sw, cc, cw, p1, Wlc, Wlw, Wrc, Wrw, bc, bw, Wh, bh, out):
    f32 = jnp.float32
    ic = 1.0 / jnp.maximum(cc[...], 1.0)
    iw = 1.0 / jnp.maximum(cw[...], 1.0)
    dot = functools.partial(lax.dot_general,
                            dimension_numbers=(((1,), (1,)), ((), ())),
                            preferred_element_type=f32)
    p2 = dot(Sc[...] * ic, Wlc[...]) + dot(Sw[...] * iw, Wlw[...])
    p2 = p2 + dot(p1[...], Wrc[...] + Wrw[...]) + (bc[...] + bw[...])[None, :]
    out[...] = dot(p2, Wh[...]) + bh[...][None, :]


def _row_spec():
    return pl.BlockSpec((BLK, 128), lambda i: (i, 0))


def _cnt_spec():
    return pl.BlockSpec((BLK, 1), lambda i: (i, 0))


def _full2(shape):
    return pl.BlockSpec(shape, lambda i: tuple(0 for _ in shape))


def kernel(x_paper, x_author, ei_cites, ei_writes, ei_written_by,
           Wl_c1, Wr_c1, b_c1, Wl_w1, Wr_w1, b_w1, Wl_b1, Wr_b1, b_b1,
           Wl_c2, Wr_c2, b_c2, Wl_w2, Wr_w2, b_w2, Wh, bh):
    zrows = jnp.zeros((ZPAD, 128), jnp.float32)
    xp_t = jnp.concatenate([x_paper, zrows])
    xa_t = jnp.concatenate([x_author, zrows])
    srcC, dstC, nmC = _pad_edges(ei_cites)
    srcW, dstW, nmW = _pad_edges(ei_writes)
    srcB, dstB, nmB = _pad_edges(ei_written_by)

    round1 = _make_sc_round([nmC, nmW, nmB], with_counts=True)
    S_c1, S_w1, S_b1, cnt_c, cnt_w, cnt_b = round1(
        xp_t, srcC, dstC, xa_t, srcW, dstW, xp_t, srcB, dstB)

    cc = cnt_c.reshape(N_DST, 1)
    cw = cnt_w.reshape(N_DST, 1)
    cb = cnt_b.reshape(N_DST, 1)

    w_spec = _full2((128, 128))
    b_spec = pl.BlockSpec((128,), lambda i: (0,))
    p1, a1 = pl.pallas_call(
        _tc1_body,
        grid=(GRID,),
        in_specs=[_row_spec(), _row_spec(), _row_spec(),
                  _cnt_spec(), _cnt_spec(), _cnt_spec(),
                  _row_spec(), _row_spec(),
                  w_spec, w_spec, w_spec, w_spec, w_spec, w_spec,
                  b_spec, b_spec, b_spec],
        out_specs=[_row_spec(), _row_spec()],
        out_shape=[jax.ShapeDtypeStruct((N_DST, 128), jnp.float32),
                   jax.ShapeDtypeStruct((N_DST, 128), jnp.float32)],
    )(S_c1, S_w1, S_b1, cc, cw, cb, x_paper, x_author,
      Wl_c1, Wl_w1, Wl_b1, Wr_c1, Wr_w1, Wr_b1, b_c1, b_w1, b_b1)

    p1_t = jnp.concatenate([p1, zrows])
    a1_t = jnp.concatenate([a1, zrows])
    round2 = _make_sc_round([nmC, nmW], with_counts=False)
    S_c2, S_w2 = round2(p1_t, srcC, dstC, a1_t, srcW, dstW)

    out = pl.pallas_call(
        _tc2_body,
        grid=(GRID,),
        in_specs=[_row_spec(), _row_spec(), _cnt_spec(), _cnt_spec(),
                  _row_spec(),
                  w_spec, w_spec, w_spec, w_spec,
                  pl.BlockSpec((128,), lambda i: (0,)),
                  pl.BlockSpec((128,), lambda i: (0,)),
                  _full2((349, 128)),
                  pl.BlockSpec((349,), lambda i: (0,))],
        out_specs=pl.BlockSpec((BLK, 349), lambda i: (i, 0)),
        out_shape=jax.ShapeDtypeStruct((N_DST, 349), jnp.float32),
    )(S_c2, S_w2, cc, cw, p1,
      Wl_c2, Wl_w2, Wr_c2, Wr_w2, b_c2, b_w2, Wh, bh)
    return out


# speedup vs baseline: 4.3035x; 1.0738x over previous
import functools

import jax
import jax.numpy as jnp
from jax import lax
from jax.experimental import pallas as pl
from jax.experimental.pallas import tpu as pltpu
from jax.experimental.pallas import tpu_sc as plsc

NC = 2
NS = 16
SUB = 128
MACRO = 2048

N_DST = 50000
RANGE = 12512
NRANGE = 4
WR = 784
WR15 = 752
WR15_LAST = 704
ZBASE = N_DST
ZPAD = 512


def _pad_edges(ei):
    E = ei.shape[1]
    ept = ((E // NS + MACRO - 1) // MACRO) * MACRO
    Ep = ept * NS
    pad = Ep - E
    idx = jnp.arange(pad, dtype=jnp.int32)
    src = jnp.concatenate([ei[0].astype(jnp.int32), ZBASE + idx % ZPAD])
    dst = jnp.concatenate([ei[1].astype(jnp.int32),
                           NRANGE * RANGE + idx % 16])
    return src, dst, ept // MACRO


def _make_sc_round(n_macros_list, with_counts):
    ntypes = len(n_macros_list)

    def body(*refs):
        k = 0
        tables = []
        srcs = []
        dsts = []
        for _ in range(ntypes):
            tables.append(refs[k]); srcs.append(refs[k + 1])
            dsts.append(refs[k + 2]); k += 3
        outs = list(refs[k:k + ntypes]); k += ntypes
        cnts = []
        if with_counts:
            cnts = list(refs[k:k + ntypes]); k += ntypes
        (acc, cacc, sbuf, dbuf, ps, pd, pdrow, ones, vt, rows,
         z1d, gsem, issem, isdem) = refs[k:]

        core = lax.axis_index("c")
        sid = lax.axis_index("s")
        lane = lax.broadcasted_iota(jnp.int32, (16,), 0)
        zv = jnp.zeros((16,), jnp.float32)
        ov = jnp.ones((16,), jnp.float32)

        def fill_rows_zero(i, _):
            for kk in range(8):
                rows[i, pl.ds(kk * 16, 16)] = zv
            return 0

        def fill_z1(i, _):
            z1d[pl.ds(i * 16, 16)] = zv
            return 0
        lax.fori_loop(0, 49, fill_z1, 0)
        for i in range(8):
            ones[pl.ds(i * 16, 16)] = ov

        for t in range(ntypes):
            n_macros = n_macros_list[t]
            tbl, src1d, dst1d, out = tables[t], srcs[t], dsts[t], outs[t]
            for rr in range(2):
                rid = core * 2 + rr
                base = rid * RANGE
                lo_off = sid * WR

                lax.fori_loop(0, 128, fill_rows_zero, 0)

                @pl.when(sid < NS - 1)
                def _():
                    for r in range(6):
                        pltpu.sync_copy(
                            rows, acc.at[pl.ds(lo_off + 128 * r, 128), :])
                    pltpu.sync_copy(rows.at[pl.ds(0, 16), :],
                                    acc.at[pl.ds(lo_off + 768, 16), :])
                    if with_counts:
                        pltpu.sync_copy(z1d, cacc.at[pl.ds(lo_off, WR)])

                @pl.when(sid == NS - 1)
                def _():
                    for r in range(5):
                        pltpu.sync_copy(
                            rows, acc.at[pl.ds(lo_off + 128 * r, 128), :])
                    pltpu.sync_copy(rows.at[pl.ds(0, 112), :],
                                    acc.at[pl.ds(lo_off + 640, 112), :])
                    if with_counts:
                        pltpu.sync_copy(z1d.at[pl.ds(0, WR15)],
                                        cacc.at[pl.ds(lo_off, WR15)])
                plsc.subcore_barrier()

                ept = n_macros * MACRO
                ebase = sid * ept

                pltpu.sync_copy(src1d.at[pl.ds(ebase, MACRO)],
                                sbuf.at[pl.ds(0, MACRO)])
                pltpu.sync_copy(dst1d.at[pl.ds(ebase, MACRO)],
                                dbuf.at[pl.ds(0, MACRO)])

                def macro_body(m, _):
                    co = (m & 1) * MACRO
                    e0 = ebase + m * MACRO

                    @pl.when(m > 0)
                    def _():
                        pltpu.make_async_copy(
                            src1d.at[pl.ds(e0, MACRO)],
                            sbuf.at[pl.ds(co, MACRO)], issem).wait()
                        pltpu.make_async_copy(
                            dst1d.at[pl.ds(e0, MACRO)],
                            dbuf.at[pl.ds(co, MACRO)], isdem).wait()

                    @pl.when(m + 1 < n_macros)
                    def _():
                        e1 = ebase + (m + 1) * MACRO
                        no = MACRO - co
                        pltpu.async_copy(src1d.at[pl.ds(e1, MACRO)],
                                         sbuf.at[pl.ds(no, MACRO)],
                                         issem)
                        pltpu.async_copy(dst1d.at[pl.ds(e1, MACRO)],
                                         dbuf.at[pl.ds(no, MACRO)],
                                         isdem)

                    def scan_body(i, wp):
                        s = sbuf[pl.ds(co + i * 16, 16)]
                        d = dbuf[pl.ds(co + i * 16, 16)]
                        lov = d - base
                        inr = (lov >= 0) & (lov < RANGE)
                        im = inr.astype(jnp.int32)
                        c = plsc.cumsum(im)
                        pos = wp + c - 1
                        plsc.store_scatter(ps, [pos], s, mask=inr)
                        plsc.store_scatter(pd, [pos], lov, mask=inr)
                        return wp + jnp.squeeze(
                            lax.slice(c, (15,), (16,)))
                    wp = lax.fori_loop(0, MACRO // 16, scan_body, 0)
                    nb = wp // 128
                    rem = wp - nb * 128

                    @pl.when(rem > 0)
                    def _():
                        for ff in range(8):
                            addr = nb * 128 + ff * 16
                            fmask = (ff * 16 + lane) >= rem
                            cs = ps[pl.ds(addr, 16)]
                            cd = pd[pl.ds(addr, 16)]
                            fs = ZBASE + ((addr + lane) & (ZPAD - 1))
                            fd = (lo_off + ff * 16 + lane) & 4095
                            ps[pl.ds(addr, 16)] = jnp.where(fmask, fs, cs)
                            pd[pl.ds(addr, 16)] = jnp.where(fmask, fd, cd)
                            if with_counts:
                                vt[pl.ds(ff * 16, 16)] = jnp.where(
                                    fmask, 0.0, 1.0)

                    def fire(b, cnt_src):
                        rp = b * 128
                        for kk in range(8):
                            pdrow[0, pl.ds(kk * 16, 16)] = (
                                pd[pl.ds(rp + kk * 16, 16)])
                        pltpu.async_copy(
                            tbl.at[ps.at[pl.ds(rp, 128)]], rows,
                            gsem).wait()
                        pltpu.sync_copy(rows, acc.at[pdrow.at[0]],
                                        add=True)
                        if with_counts:
                            pltpu.sync_copy(cnt_src,
                                            cacc.at[pdrow.at[0]],
                                            add=True)

                    def fire_body(b, _):
                        fire(b, ones)
                        return 0
                    lax.fori_loop(0, nb, fire_body, 0)

                    @pl.when(rem > 0)
                    def _():
                        fire(nb, vt)
                    return 0
                lax.fori_loop(0, n_macros, macro_body, 0)
                plsc.subcore_barrier()

                gbase = base + lo_off

                @pl.when(sid < NS - 1)
                def _():
                    pltpu.sync_copy(acc.at[pl.ds(lo_off, WR), :],
                                    out.at[pl.ds(gbase, WR), :])
                    if with_counts:
                        pltpu.sync_copy(cacc.at[pl.ds(lo_off, WR)], z1d)
                        pltpu.sync_copy(z1d,
                                        cnts[t].at[pl.ds(gbase, WR)])

                @pl.when((sid == NS - 1) & (rid < NRANGE - 1))
                def _():
                    pltpu.sync_copy(acc.at[pl.ds(lo_off, WR15), :],
                                    out.at[pl.ds(gbase, WR15), :])
                    if with_counts:
                        pltpu.sync_copy(cacc.at[pl.ds(lo_off, WR15)],
                                        z1d.at[pl.ds(0, WR15)])
                        pltpu.sync_copy(z1d.at[pl.ds(0, WR15)],
                                        cnts[t].at[pl.ds(gbase, WR15)])

                @pl.when((sid == NS - 1) & (rid == NRANGE - 1))
                def _():
                    pltpu.sync_copy(acc.at[pl.ds(lo_off, WR15_LAST), :],
                                    out.at[pl.ds(gbase, WR15_LAST), :])
                    if with_counts:
                        pltpu.sync_copy(
                            cacc.at[pl.ds(lo_off, WR15_LAST)],
                            z1d.at[pl.ds(0, WR15_LAST)])
                        pltpu.sync_copy(
                            z1d.at[pl.ds(0, WR15_LAST)],
                            cnts[t].at[pl.ds(gbase, WR15_LAST)])

                if with_counts:
                    lax.fori_loop(0, 49, fill_z1, 0)

    out_type = [jax.ShapeDtypeStruct((N_DST, 128), jnp.float32)
                for _ in range(ntypes)]
    if with_counts:
        out_type += [jax.ShapeDtypeStruct((N_DST,), jnp.float32)
                     for _ in range(ntypes)]

    mesh = plsc.VectorSubcoreMesh(core_axis_name="c", subcore_axis_name="s",
                                  num_cores=NC, num_subcores=NS)
    return pl.kernel(
        body,
        out_type=out_type,
        mesh=mesh,
        compiler_params=pltpu.CompilerParams(needs_layout_passes=False),
        scratch_types=[
            pltpu.VMEM_SHARED((RANGE, 128), jnp.float32),
            pltpu.VMEM_SHARED((RANGE,), jnp.float32),
            pltpu.VMEM((2 * MACRO,), jnp.int32),
            pltpu.VMEM((2 * MACRO,), jnp.int32),
            pltpu.VMEM((MACRO,), jnp.int32),
            pltpu.VMEM((MACRO,), jnp.int32),
            pltpu.VMEM((1, 128), jnp.int32),
            pltpu.VMEM((128,), jnp.float32),
            pltpu.VMEM((128,), jnp.float32),
            pltpu.VMEM((128, 128), jnp.float32),
            pltpu.VMEM((784,), jnp.float32),
            pltpu.SemaphoreType.DMA,
            pltpu.SemaphoreType.DMA,
            pltpu.SemaphoreType.DMA,
        ],
    )


BLK = 1000
GRID = N_DST // BLK


def _tc1_body(Sc, Sw, Sb, cc, cw, cb, xp, xa,
              Wlc, Wlw, Wlb, Wrc, Wrw, Wrb, bc, bw, bb, p1, a1):
    f32 = jnp.float32
    ic = 1.0 / jnp.maximum(cc[...], 1.0)
    iw = 1.0 / jnp.maximum(cw[...], 1.0)
    ib = 1.0 / jnp.maximum(cb[...], 1.0)
    dot = functools.partial(lax.dot_general,
                            dimension_numbers=(((1,), (1,)), ((), ())),
                            preferred_element_type=f32)
    p = dot(Sc[...] * ic, Wlc[...]) + dot(Sw[...] * iw, Wlw[...])
    p = p + dot(xp[...], Wrc[...] + Wrw[...])
    p1[...] = p + (bc[...] + bw[...])[None, :]
    a = dot(Sb[...] * ib, Wlb[...]) + dot(xa[...], Wrb[...])
    a1[...] = a + bb[...][None, :]


def _tc2_body(Sc, Sw, cc, cw, p1, Wlc, Wlw, Wrc, Wrw, bc, bw, Wh, bh, out):
    f32 = jnp.float32
    ic = 1.0 / jnp.maximum(cc[...], 1.0)
    iw = 1.0 / jnp.maximum(cw[...], 1.0)
    dot = functools.partial(lax.dot_general,
                            dimension_numbers=(((1,), (1,)), ((), ())),
                            preferred_element_type=f32)
    p2 = dot(Sc[...] * ic, Wlc[...]) + dot(Sw[...] * iw, Wlw[...])
    p2 = p2 + dot(p1[...], Wrc[...] + Wrw[...]) + (bc[...] + bw[...])[None, :]
    out[...] = dot(p2, Wh[...]) + bh[...][None, :]


def _row_spec():
    return pl.BlockSpec((BLK, 128), lambda i: (i, 0))


def _cnt_spec():
    return pl.BlockSpec((BLK, 1), lambda i: (i, 0))


def _full2(shape):
    return pl.BlockSpec(shape, lambda i: tuple(0 for _ in shape))


def kernel(x_paper, x_author, ei_cites, ei_writes, ei_written_by,
           Wl_c1, Wr_c1, b_c1, Wl_w1, Wr_w1, b_w1, Wl_b1, Wr_b1, b_b1,
           Wl_c2, Wr_c2, b_c2, Wl_w2, Wr_w2, b_w2, Wh, bh):
    zrows = jnp.zeros((ZPAD, 128), jnp.float32)
    xp_t = jnp.concatenate([x_paper, zrows])
    xa_t = jnp.concatenate([x_author, zrows])
    srcC, dstC, nmC = _pad_edges(ei_cites)
    srcW, dstW, nmW = _pad_edges(ei_writes)
    srcB, dstB, nmB = _pad_edges(ei_written_by)

    round1 = _make_sc_round([nmC, nmW, nmB], with_counts=True)
    S_c1, S_w1, S_b1, cnt_c, cnt_w, cnt_b = round1(
        xp_t, srcC, dstC, xa_t, srcW, dstW, xp_t, srcB, dstB)

    cc = cnt_c.reshape(N_DST, 1)
    cw = cnt_w.reshape(N_DST, 1)
    cb = cnt_b.reshape(N_DST, 1)

    w_spec = _full2((128, 128))
    b_spec = pl.BlockSpec((128,), lambda i: (0,))
    p1, a1 = pl.pallas_call(
        _tc1_body,
        grid=(GRID,),
        in_specs=[_row_spec(), _row_spec(), _row_spec(),
                  _cnt_spec(), _cnt_spec(), _cnt_spec(),
                  _row_spec(), _row_spec(),
                  w_spec, w_spec, w_spec, w_spec, w_spec, w_spec,
                  b_spec, b_spec, b_spec],
        out_specs=[_row_spec(), _row_spec()],
        out_shape=[jax.ShapeDtypeStruct((N_DST, 128), jnp.float32),
                   jax.ShapeDtypeStruct((N_DST, 128), jnp.float32)],
    )(S_c1, S_w1, S_b1, cc, cw, cb, x_paper, x_author,
      Wl_c1, Wl_w1, Wl_b1, Wr_c1, Wr_w1, Wr_b1, b_c1, b_w1, b_b1)

    p1_t = jnp.concatenate([p1, zrows])
    a1_t = jnp.concatenate([a1, zrows])
    round2 = _make_sc_round([nmC, nmW], with_counts=False)
    S_c2, S_w2 = round2(p1_t, srcC, dstC, a1_t, srcW, dstW)

    out = pl.pallas_call(
        _tc2_body,
        grid=(GRID,),
        in_specs=[_row_spec(), _row_spec(), _cnt_spec(), _cnt_spec(),
                  _row_spec(),
                  w_spec, w_spec, w_spec, w_spec,
                  pl.BlockSpec((128,), lambda i: (0,)),
                  pl.BlockSpec((128,), lambda i: (0,)),
                  _full2((349, 128)),
                  pl.BlockSpec((349,), lambda i: (0,))],
        out_specs=pl.BlockSpec((BLK, 349), lambda i: (i, 0)),
        out_shape=jax.ShapeDtypeStruct((N_DST, 349), jnp.float32),
    )(S_c2, S_w2, cc, cw, p1,
      Wl_c2, Wl_w2, Wr_c2, Wr_w2, b_c2, b_w2, Wh, bh)
    return out
